# Initial kernel scaffold; baseline (speedup 1.0000x reference)
import jax
import jax.numpy as jnp
from jax.experimental import pallas as pl


def kernel(x, table, Wih0f, Whh0f, bih0f, bhh0f, Wih0b, Whh0b, bih0b, bhh0b, Wih1f, Whh1f, bih1f, bhh1f, Wih1b, Whh1b, bih1b, bhh1b, fcW, fcb):
    raise NotImplementedError("write your pallas kernel here")


# speedup vs baseline: 2.8303x; 2.8303x over previous
import jax
import jax.numpy as jnp
from jax.experimental import pallas as pl
from jax.experimental.pallas import tpu as pltpu
from jax.experimental.pallas import tpu_sc as plsc

EMB = 128
HID = 512
B = 1024
T = 200

_GATHER_WINDOW = 128
_N_IDX = B * T


def _sc_gather(table, idx):
    n = _N_IDX
    e = table.shape[1]
    mesh = plsc.VectorSubcoreMesh(core_axis_name="core", subcore_axis_name="subcore")

    @pl.kernel(out_type=jax.ShapeDtypeStruct((n, e), table.dtype), mesh=mesh)
    def gather_kernel(tab_hbm, i_hbm, o_hbm):
        def body(i_vmem, o_vmem):
            pltpu.sync_copy(tab_hbm.at[i_vmem.at[0]], o_vmem)

        pltpu.emit_pipeline(
            body,
            grid=(n // _GATHER_WINDOW,),
            in_specs=[pl.BlockSpec((1, _GATHER_WINDOW), index_map=lambda i: (0, i))],
            out_specs=[pl.BlockSpec((_GATHER_WINDOW, e), index_map=lambda i: (i, 0))],
            core_axis_name=("core", "subcore"),
            dimension_semantics=(pltpu.PARALLEL,),
        )(i_hbm, o_hbm)

    return gather_kernel(table, idx)


def _lstm_step(x_parts, wih_refs, whh_ref, b_ref, h_ref, c_ref):
    g = jnp.dot(x_parts[0], wih_refs[0][...], preferred_element_type=jnp.float32)
    for xp, wr in zip(x_parts[1:], wih_refs[1:]):
        g = g + jnp.dot(xp, wr[...], preferred_element_type=jnp.float32)
    g = g + jnp.dot(h_ref[...].astype(jnp.bfloat16), whh_ref[...],
                    preferred_element_type=jnp.float32)
    g = g + b_ref[...]
    i = jax.nn.sigmoid(g[:, :HID])
    f = jax.nn.sigmoid(g[:, HID:2 * HID])
    gg = jnp.tanh(g[:, 2 * HID:3 * HID])
    o = jax.nn.sigmoid(g[:, 3 * HID:])
    c2 = f * c_ref[...] + i * gg
    h2 = o * jnp.tanh(c2)
    c_ref[...] = c2
    h_ref[...] = h2
    return h2


def _l0_body(xf_ref, xb_ref, wif_ref, whf_ref, bf_ref, wib_ref, whb_ref, bb_ref,
             hfo_ref, hbo_ref, hf, cf, hb, cb):
    t = pl.program_id(0)

    @pl.when(t == 0)
    def _():
        hf[...] = jnp.zeros_like(hf)
        cf[...] = jnp.zeros_like(cf)
        hb[...] = jnp.zeros_like(hb)
        cb[...] = jnp.zeros_like(cb)

    h2f = _lstm_step((xf_ref[0].astype(jnp.bfloat16),), (wif_ref,), whf_ref,
                     bf_ref, hf, cf)
    hfo_ref[0] = h2f.astype(jnp.bfloat16)
    h2b = _lstm_step((xb_ref[0].astype(jnp.bfloat16),), (wib_ref,), whb_ref,
                     bb_ref, hb, cb)
    hbo_ref[0] = h2b.astype(jnp.bfloat16)


def _full_spec(a):
    nd = a.ndim
    return pl.BlockSpec(a.shape, lambda t, _n=nd: (0,) * _n)


def _bilstm_layer0(emb, wifT, whfT, bf, wibT, whbT, bb):
    out_shape = [jax.ShapeDtypeStruct((T, B, HID), jnp.bfloat16),
                 jax.ShapeDtypeStruct((T, B, HID), jnp.bfloat16)]
    return pl.pallas_call(
        _l0_body,
        grid=(T,),
        in_specs=[
            pl.BlockSpec((1, B, EMB), lambda t: (t, 0, 0)),
            pl.BlockSpec((1, B, EMB), lambda t: (T - 1 - t, 0, 0)),
            _full_spec(wifT), _full_spec(whfT), _full_spec(bf),
            _full_spec(wibT), _full_spec(whbT), _full_spec(bb),
        ],
        out_specs=[
            pl.BlockSpec((1, B, HID), lambda t: (t, 0, 0)),
            pl.BlockSpec((1, B, HID), lambda t: (T - 1 - t, 0, 0)),
        ],
        out_shape=out_shape,
        scratch_shapes=[pltpu.VMEM((B, HID), jnp.float32) for _ in range(4)],
        compiler_params=pltpu.CompilerParams(dimension_semantics=("arbitrary",)),
    )(emb, emb, wifT, whfT, bf, wibT, whbT, bb)


def _l1_body(hff_ref, hbf_ref, hfb_ref, hbb_ref,
             wf1_ref, wf2_ref, whf_ref, bf_ref,
             wb1_ref, wb2_ref, whb_ref, bb_ref,
             fwf_ref, fwb_ref, fcb_ref,
             out_ref, hf, cf, hb, cb):
    t = pl.program_id(0)

    @pl.when(t == 0)
    def _():
        hf[...] = jnp.zeros_like(hf)
        cf[...] = jnp.zeros_like(cf)
        hb[...] = jnp.zeros_like(hb)
        cb[...] = jnp.zeros_like(cb)

    _lstm_step((hff_ref[0], hbf_ref[0]), (wf1_ref, wf2_ref), whf_ref, bf_ref, hf, cf)
    _lstm_step((hfb_ref[0], hbb_ref[0]), (wb1_ref, wb2_ref), whb_ref, bb_ref, hb, cb)

    @pl.when(t == T - 1)
    def _():
        vf = jnp.sum(hf[...] * fwf_ref[...], axis=1, keepdims=True)
        vb = jnp.sum(hb[...] * fwb_ref[...], axis=1, keepdims=True)
        out_ref[...] = jax.nn.sigmoid(vf + vb + fcb_ref[...])


def _bilstm_layer1_fc(hf0, hb0, wf1, wf2, whf, bf, wb1, wb2, whb, bb, fwf, fwb, fcb):
    seq_spec_f = pl.BlockSpec((1, B, HID), lambda t: (t, 0, 0))
    seq_spec_b = pl.BlockSpec((1, B, HID), lambda t: (T - 1 - t, 0, 0))
    return pl.pallas_call(
        _l1_body,
        grid=(T,),
        in_specs=[
            seq_spec_f, seq_spec_f, seq_spec_b, seq_spec_b,
            _full_spec(wf1), _full_spec(wf2), _full_spec(whf), _full_spec(bf),
            _full_spec(wb1), _full_spec(wb2), _full_spec(whb), _full_spec(bb),
            _full_spec(fwf), _full_spec(fwb), _full_spec(fcb),
        ],
        out_specs=pl.BlockSpec((B, 1), lambda t: (0, 0)),
        out_shape=jax.ShapeDtypeStruct((B, 1), jnp.float32),
        scratch_shapes=[pltpu.VMEM((B, HID), jnp.float32) for _ in range(4)],
        compiler_params=pltpu.CompilerParams(dimension_semantics=("arbitrary",)),
    )(hf0, hb0, hf0, hb0, wf1, wf2, whf, bf, wb1, wb2, whb, bb, fwf, fwb, fcb)


def kernel(x, table, Wih0f, Whh0f, bih0f, bhh0f, Wih0b, Whh0b, bih0b, bhh0b,
           Wih1f, Whh1f, bih1f, bhh1f, Wih1b, Whh1b, bih1b, bhh1b, fcW, fcb):
    bf16 = jnp.bfloat16

    idx = x.astype(jnp.int32).T.reshape(1, _N_IDX)
    emb = _sc_gather(table, idx).reshape(T, B, EMB)

    wif0 = Wih0f.T.astype(bf16)
    whf0 = Whh0f.T.astype(bf16)
    b0f = (bih0f + bhh0f).reshape(1, 4 * HID)
    wib0 = Wih0b.T.astype(bf16)
    whb0 = Whh0b.T.astype(bf16)
    b0b = (bih0b + bhh0b).reshape(1, 4 * HID)

    hf0, hb0 = _bilstm_layer0(emb, wif0, whf0, b0f, wib0, whb0, b0b)

    w1f = Wih1f.T.astype(bf16)
    w1b = Wih1b.T.astype(bf16)
    wf1, wf2 = w1f[:HID], w1f[HID:]
    wb1, wb2 = w1b[:HID], w1b[HID:]
    whf1 = Whh1f.T.astype(bf16)
    whb1 = Whh1b.T.astype(bf16)
    b1f = (bih1f + bhh1f).reshape(1, 4 * HID)
    b1b = (bih1b + bhh1b).reshape(1, 4 * HID)

    fwf = fcW[:, :HID]
    fwb = fcW[:, HID:]
    fcbr = fcb.reshape(1, 1)

    return _bilstm_layer1_fc(hf0, hb0, wf1, wf2, whf1, b1f, wb1, wb2, whb1, b1b,
                             fwf, fwb, fcbr)


# speedup vs baseline: 2.9430x; 1.0398x over previous
import jax
import jax.numpy as jnp
from jax.experimental import pallas as pl
from jax.experimental.pallas import tpu as pltpu
from jax.experimental.pallas import tpu_sc as plsc

EMB = 128
HID = 512
B = 1024
T = 200

_GATHER_WINDOW = 128
_N_IDX = B * T


def _sc_gather(table, idx):
    n = _N_IDX
    e = table.shape[1]
    mesh = plsc.VectorSubcoreMesh(core_axis_name="core", subcore_axis_name="subcore")

    @pl.kernel(out_type=jax.ShapeDtypeStruct((n, e), table.dtype), mesh=mesh)
    def gather_kernel(tab_hbm, i_hbm, o_hbm):
        def body(i_vmem, o_vmem):
            pltpu.sync_copy(tab_hbm.at[i_vmem.at[0]], o_vmem)

        pltpu.emit_pipeline(
            body,
            grid=(n // _GATHER_WINDOW,),
            in_specs=[pl.BlockSpec((1, _GATHER_WINDOW), index_map=lambda i: (0, i))],
            out_specs=[pl.BlockSpec((_GATHER_WINDOW, e), index_map=lambda i: (i, 0))],
            core_axis_name=("core", "subcore"),
            dimension_semantics=(pltpu.PARALLEL,),
        )(i_hbm, o_hbm)

    return gather_kernel(table, idx)


def _gates(g, c_ref, h_ref):
    i = jax.nn.sigmoid(g[:, :HID])
    f = jax.nn.sigmoid(g[:, HID:2 * HID])
    gg = jnp.tanh(g[:, 2 * HID:3 * HID])
    o = jax.nn.sigmoid(g[:, 3 * HID:])
    c2 = f * c_ref[...] + i * gg
    h2 = o * jnp.tanh(c2)
    c_ref[...] = c2
    hb = h2.astype(jnp.bfloat16)
    h_ref[...] = hb
    return hb


def _l0_body(xf_ref, xb_ref, wf_ref, whf_ref, bf_ref, wb_ref, whb_ref, bb_ref,
             hfo_ref, hbo_ref, hf, cf, hb, cb):
    t = pl.program_id(0)

    @pl.when(t == 0)
    def _():
        hf[...] = jnp.zeros_like(hf)
        cf[...] = jnp.zeros_like(cf)
        hb[...] = jnp.zeros_like(hb)
        cb[...] = jnp.zeros_like(cb)

    def step(x_ref, wih_ref, whh_ref, b_ref, h, c, out_ref):
        g = jnp.dot(h[...], whh_ref[...], preferred_element_type=jnp.float32)
        g = g + jnp.dot(x_ref[0].astype(jnp.bfloat16), wih_ref[...],
                        preferred_element_type=jnp.float32)
        g = g + b_ref[...]
        out_ref[0] = _gates(g, c, h)

    step(xf_ref, wf_ref, whf_ref, bf_ref, hf, cf, hfo_ref)
    step(xb_ref, wb_ref, whb_ref, bb_ref, hb, cb, hbo_ref)


def _full_spec(a):
    nd = a.ndim
    return pl.BlockSpec(a.shape, lambda t, _n=nd: (0,) * _n)


def _bilstm_layer0(emb, wifT, whfT, bf, wibT, whbT, bb):
    out_shape = [jax.ShapeDtypeStruct((T, B, HID), jnp.bfloat16),
                 jax.ShapeDtypeStruct((T, B, HID), jnp.bfloat16)]
    return pl.pallas_call(
        _l0_body,
        grid=(T,),
        in_specs=[
            pl.BlockSpec((1, B, EMB), lambda t: (t, 0, 0)),
            pl.BlockSpec((1, B, EMB), lambda t: (T - 1 - t, 0, 0)),
            _full_spec(wifT), _full_spec(whfT), _full_spec(bf),
            _full_spec(wibT), _full_spec(whbT), _full_spec(bb),
        ],
        out_specs=[
            pl.BlockSpec((1, B, HID), lambda t: (t, 0, 0)),
            pl.BlockSpec((1, B, HID), lambda t: (T - 1 - t, 0, 0)),
        ],
        out_shape=out_shape,
        scratch_shapes=[pltpu.VMEM((B, HID), jnp.bfloat16),
                        pltpu.VMEM((B, HID), jnp.float32),
                        pltpu.VMEM((B, HID), jnp.bfloat16),
                        pltpu.VMEM((B, HID), jnp.float32)],
        compiler_params=pltpu.CompilerParams(dimension_semantics=("arbitrary",)),
    )(emb, emb, wifT, whfT, bf, wibT, whbT, bb)


def _l1_body(hff_ref, hbf_ref, hfb_ref, hbb_ref,
             wf_ref, whf_ref, bf_ref,
             wb_ref, whb_ref, bb_ref,
             fwf_ref, fwb_ref, fcb_ref,
             out_ref, hf, cf, hb, cb, xcf, xcb):
    t = pl.program_id(0)

    @pl.when(t == 0)
    def _():
        hf[...] = jnp.zeros_like(hf)
        cf[...] = jnp.zeros_like(cf)
        hb[...] = jnp.zeros_like(hb)
        cb[...] = jnp.zeros_like(cb)

    def step(in1_ref, in2_ref, wih_ref, whh_ref, b_ref, h, c, xc):
        g = jnp.dot(h[...], whh_ref[...], preferred_element_type=jnp.float32)
        xc[:, :HID] = in1_ref[0]
        xc[:, HID:] = in2_ref[0]
        g = g + jnp.dot(xc[...], wih_ref[...], preferred_element_type=jnp.float32)
        g = g + b_ref[...]
        _gates(g, c, h)

    step(hff_ref, hbf_ref, wf_ref, whf_ref, bf_ref, hf, cf, xcf)
    step(hfb_ref, hbb_ref, wb_ref, whb_ref, bb_ref, hb, cb, xcb)

    @pl.when(t == T - 1)
    def _():
        vf = jnp.sum(hf[...].astype(jnp.float32) * fwf_ref[...], axis=1,
                     keepdims=True)
        vb = jnp.sum(hb[...].astype(jnp.float32) * fwb_ref[...], axis=1,
                     keepdims=True)
        out_ref[...] = jax.nn.sigmoid(vf + vb + fcb_ref[...])


def _bilstm_layer1_fc(hf0, hb0, w1f, whf, bf, w1b, whb, bb, fwf, fwb, fcb):
    seq_spec_f = pl.BlockSpec((1, B, HID), lambda t: (t, 0, 0))
    seq_spec_b = pl.BlockSpec((1, B, HID), lambda t: (T - 1 - t, 0, 0))
    return pl.pallas_call(
        _l1_body,
        grid=(T,),
        in_specs=[
            seq_spec_f, seq_spec_f, seq_spec_b, seq_spec_b,
            _full_spec(w1f), _full_spec(whf), _full_spec(bf),
            _full_spec(w1b), _full_spec(whb), _full_spec(bb),
            _full_spec(fwf), _full_spec(fwb), _full_spec(fcb),
        ],
        out_specs=pl.BlockSpec((B, 1), lambda t: (0, 0)),
        out_shape=jax.ShapeDtypeStruct((B, 1), jnp.float32),
        scratch_shapes=[pltpu.VMEM((B, HID), jnp.bfloat16),
                        pltpu.VMEM((B, HID), jnp.float32),
                        pltpu.VMEM((B, HID), jnp.bfloat16),
                        pltpu.VMEM((B, HID), jnp.float32),
                        pltpu.VMEM((B, 2 * HID), jnp.bfloat16),
                        pltpu.VMEM((B, 2 * HID), jnp.bfloat16)],
        compiler_params=pltpu.CompilerParams(dimension_semantics=("arbitrary",)),
    )(hf0, hb0, hf0, hb0, w1f, whf, bf, w1b, whb, bb, fwf, fwb, fcb)


def kernel(x, table, Wih0f, Whh0f, bih0f, bhh0f, Wih0b, Whh0b, bih0b, bhh0b,
           Wih1f, Whh1f, bih1f, bhh1f, Wih1b, Whh1b, bih1b, bhh1b, fcW, fcb):
    bf16 = jnp.bfloat16

    idx = x.astype(jnp.int32).T.reshape(1, _N_IDX)
    emb = _sc_gather(table, idx).reshape(T, B, EMB)

    wif0 = Wih0f.T.astype(bf16)
    whf0 = Whh0f.T.astype(bf16)
    b0f = (bih0f + bhh0f).reshape(1, 4 * HID)
    wib0 = Wih0b.T.astype(bf16)
    whb0 = Whh0b.T.astype(bf16)
    b0b = (bih0b + bhh0b).reshape(1, 4 * HID)

    hf0, hb0 = _bilstm_layer0(emb, wif0, whf0, b0f, wib0, whb0, b0b)

    w1f = Wih1f.T.astype(bf16)
    w1b = Wih1b.T.astype(bf16)
    whf1 = Whh1f.T.astype(bf16)
    whb1 = Whh1b.T.astype(bf16)
    b1f = (bih1f + bhh1f).reshape(1, 4 * HID)
    b1b = (bih1b + bhh1b).reshape(1, 4 * HID)

    fwf = fcW[:, :HID]
    fwb = fcW[:, HID:]
    fcbr = fcb.reshape(1, 1)

    return _bilstm_layer1_fc(hf0, hb0, w1f, whf1, b1f, w1b, whb1, b1b,
                             fwf, fwb, fcbr)
